# baseline (speedup 1.0000x reference)
import functools

import jax
import jax.numpy as jnp
from jax import lax
from jax.experimental import pallas as pl
from jax.experimental.pallas import tpu as pltpu
from jax.experimental.pallas import tpu_sc as plsc

B = 65536
IN_FEAT = 128
HIDDEN = 128
OUT_FEAT = 128
CODEBOOK_DIM = 32
NUM_CODES = 256

BLOCK_B = 8192

NUM_CORES = 2
NUM_SUBCORES = 16
NUM_WORKERS = NUM_CORES * NUM_SUBCORES
B_PER_WORKER = B // NUM_WORKERS
CHUNK = 512


def _fold_kernel(cb_ref, wo_ref, bo_ref, wu_ref, bu_ref, table_ref):
    t = jnp.dot(cb_ref[...], wo_ref[...],
                preferred_element_type=jnp.float32) + bo_ref[...]
    t = jnp.dot(t, wu_ref[...],
                preferred_element_type=jnp.float32) + bu_ref[...]
    table_ref[...] = jnp.clip(t, -1.0, 1.0)


def _idx_kernel(x_ref, wd_ref, bd_ref, wi_ref, bi_ref, cb_ref, idx_ref):
    h = jnp.dot(x_ref[...], wd_ref[...],
                preferred_element_type=jnp.float32) + bd_ref[...]
    z = jnp.dot(h, wi_ref[...],
                preferred_element_type=jnp.float32) + bi_ref[...]
    cb = cb_ref[...]
    zc = jnp.dot(z, cb.T, preferred_element_type=jnp.float32)
    s = zc - 0.5 * jnp.sum(cb * cb, axis=-1)[None, :]
    idx_ref[...] = jnp.argmax(s, axis=-1).astype(jnp.int32)[:, None]


_SC_MESH = plsc.VectorSubcoreMesh(core_axis_name="c", subcore_axis_name="s")


@functools.partial(
    pl.kernel,
    mesh=_SC_MESH,
    out_type=jax.ShapeDtypeStruct((B, OUT_FEAT), jnp.float32),
    scratch_types=[
        pltpu.VMEM((CHUNK,), jnp.int32),
        pltpu.VMEM((CHUNK, OUT_FEAT), jnp.float32),
        pltpu.SemaphoreType.DMA,
    ],
)
def _sc_gather(table_hbm, idx_hbm, out_hbm, idx_v, rows_v, sem):
    wid = lax.axis_index("s") * NUM_CORES + lax.axis_index("c")
    base = wid * B_PER_WORKER
    for c in range(B_PER_WORKER // CHUNK):
        off = base + c * CHUNK
        pltpu.sync_copy(idx_hbm.at[pl.ds(off, CHUNK)], idx_v)
        pltpu.async_copy(table_hbm.at[idx_v], rows_v, sem).wait()
        pltpu.sync_copy(rows_v, out_hbm.at[pl.ds(off, CHUNK)])


def kernel(x, W_down, b_down, W_in, b_in, codebook, W_out, b_out, W_up, b_up):
    table = pl.pallas_call(
        _fold_kernel,
        out_shape=jax.ShapeDtypeStruct((NUM_CODES, OUT_FEAT), jnp.float32),
    )(codebook, W_out, b_out, W_up, b_up)

    grid = (B // BLOCK_B,)
    idx2d = pl.pallas_call(
        _idx_kernel,
        grid=grid,
        in_specs=[
            pl.BlockSpec((BLOCK_B, IN_FEAT), lambda i: (i, 0)),
            pl.BlockSpec((IN_FEAT, HIDDEN), lambda i: (0, 0)),
            pl.BlockSpec((HIDDEN,), lambda i: (0,)),
            pl.BlockSpec((HIDDEN, CODEBOOK_DIM), lambda i: (0, 0)),
            pl.BlockSpec((CODEBOOK_DIM,), lambda i: (0,)),
            pl.BlockSpec((NUM_CODES, CODEBOOK_DIM), lambda i: (0, 0)),
        ],
        out_specs=pl.BlockSpec((BLOCK_B, 1), lambda i: (i, 0)),
        out_shape=jax.ShapeDtypeStruct((B, 1), jnp.int32),
        compiler_params=pltpu.CompilerParams(
            dimension_semantics=("parallel",),
        ),
    )(x, W_down, b_down, W_in, b_in, codebook)

    indices = idx2d.reshape(B)
    out = _sc_gather(table, indices)
    commit_loss = jnp.zeros((), jnp.float32)
    return out, indices, commit_loss


# speedup vs baseline: 7.6854x; 7.6854x over previous
import functools

import jax
import jax.numpy as jnp
from jax import lax
from jax.experimental import pallas as pl
from jax.experimental.pallas import tpu as pltpu
from jax.experimental.pallas import tpu_sc as plsc

B = 65536
IN_FEAT = 128
HIDDEN = 128
OUT_FEAT = 128
CODEBOOK_DIM = 32
NUM_CODES = 256

BLOCK_B = 8192

NUM_CORES = 2
NUM_SUBCORES = 16
NUM_WORKERS = NUM_CORES * NUM_SUBCORES
B_PER_WORKER = B // NUM_WORKERS
CHUNK = 512


def _fold_kernel(cb_ref, wo_ref, bo_ref, wu_ref, bu_ref, table_ref):
    t = jnp.dot(cb_ref[...], wo_ref[...],
                preferred_element_type=jnp.float32) + bo_ref[...]
    t = jnp.dot(t, wu_ref[...],
                preferred_element_type=jnp.float32) + bu_ref[...]
    table_ref[...] = jnp.clip(t, -1.0, 1.0)


def _idx_kernel(x_ref, wd_ref, bd_ref, wi_ref, bi_ref, cb_ref, idx_ref):
    h = jnp.dot(x_ref[...], wd_ref[...],
                preferred_element_type=jnp.float32) + bd_ref[...]
    z = jnp.dot(h, wi_ref[...],
                preferred_element_type=jnp.float32) + bi_ref[...]
    cb = cb_ref[...]
    zc = jnp.dot(z, cb.T, preferred_element_type=jnp.float32)
    s = zc - 0.5 * jnp.sum(cb * cb, axis=-1)[None, :]
    idx_ref[...] = jnp.argmax(s, axis=-1).astype(jnp.int32)[:, None]


_SC_MESH = plsc.VectorSubcoreMesh(core_axis_name="c", subcore_axis_name="s")


@functools.partial(
    pl.kernel,
    mesh=_SC_MESH,
    out_type=jax.ShapeDtypeStruct((B, OUT_FEAT), jnp.float32),
    scratch_types=[
        pltpu.VMEM((CHUNK,), jnp.int32),
        pltpu.VMEM((CHUNK, OUT_FEAT), jnp.float32),
        pltpu.VMEM_SHARED((NUM_CODES, OUT_FEAT), jnp.float32),
        pltpu.SemaphoreType.DMA,
    ],
)
def _sc_gather(table_hbm, idx_hbm, out_hbm, idx_v, rows_v, table_sh, sem):
    @pl.when(lax.axis_index("s") == 0)
    def _stage():
        pltpu.sync_copy(table_hbm, table_sh)

    plsc.subcore_barrier()
    wid = lax.axis_index("s") * NUM_CORES + lax.axis_index("c")
    base = wid * B_PER_WORKER
    for c in range(B_PER_WORKER // CHUNK):
        off = base + c * CHUNK
        pltpu.sync_copy(idx_hbm.at[pl.ds(off, CHUNK)], idx_v)
        pltpu.async_copy(table_sh.at[idx_v], rows_v, sem).wait()
        pltpu.sync_copy(rows_v, out_hbm.at[pl.ds(off, CHUNK)])


def kernel(x, W_down, b_down, W_in, b_in, codebook, W_out, b_out, W_up, b_up):
    table = pl.pallas_call(
        _fold_kernel,
        out_shape=jax.ShapeDtypeStruct((NUM_CODES, OUT_FEAT), jnp.float32),
    )(codebook, W_out, b_out, W_up, b_up)

    grid = (B // BLOCK_B,)
    idx2d = pl.pallas_call(
        _idx_kernel,
        grid=grid,
        in_specs=[
            pl.BlockSpec((BLOCK_B, IN_FEAT), lambda i: (i, 0)),
            pl.BlockSpec((IN_FEAT, HIDDEN), lambda i: (0, 0)),
            pl.BlockSpec((HIDDEN,), lambda i: (0,)),
            pl.BlockSpec((HIDDEN, CODEBOOK_DIM), lambda i: (0, 0)),
            pl.BlockSpec((CODEBOOK_DIM,), lambda i: (0,)),
            pl.BlockSpec((NUM_CODES, CODEBOOK_DIM), lambda i: (0, 0)),
        ],
        out_specs=pl.BlockSpec((BLOCK_B, 1), lambda i: (i, 0)),
        out_shape=jax.ShapeDtypeStruct((B, 1), jnp.int32),
        compiler_params=pltpu.CompilerParams(
            dimension_semantics=("parallel",),
        ),
    )(x, W_down, b_down, W_in, b_in, codebook)

    indices = idx2d.reshape(B)
    out = _sc_gather(table, indices)
    commit_loss = jnp.zeros((), jnp.float32)
    return out, indices, commit_loss


# speedup vs baseline: 8.0104x; 1.0423x over previous
import functools

import jax
import jax.numpy as jnp
from jax import lax
from jax.experimental import pallas as pl
from jax.experimental.pallas import tpu as pltpu
from jax.experimental.pallas import tpu_sc as plsc

B = 65536
IN_FEAT = 128
HIDDEN = 128
OUT_FEAT = 128
CODEBOOK_DIM = 32
NUM_CODES = 256

BLOCK_B = 8192

NUM_CORES = 2
NUM_SUBCORES = 16
NUM_WORKERS = NUM_CORES * NUM_SUBCORES
B_PER_WORKER = B // NUM_WORKERS
CHUNK = 128
N_CHUNKS = B_PER_WORKER // CHUNK


def _fold_kernel(cb_ref, wo_ref, bo_ref, wu_ref, bu_ref, table_ref):
    t = jnp.dot(cb_ref[...], wo_ref[...],
                preferred_element_type=jnp.float32) + bo_ref[...]
    t = jnp.dot(t, wu_ref[...],
                preferred_element_type=jnp.float32) + bu_ref[...]
    table_ref[...] = jnp.clip(t, -1.0, 1.0)


def _idx_kernel(x_ref, wd_ref, bd_ref, wi_ref, bi_ref, cb_ref, idx_ref):
    h = jnp.dot(x_ref[...], wd_ref[...],
                preferred_element_type=jnp.float32) + bd_ref[...]
    z = jnp.dot(h, wi_ref[...],
                preferred_element_type=jnp.float32) + bi_ref[...]
    cb = cb_ref[...]
    zc = jnp.dot(z, cb.T, preferred_element_type=jnp.float32)
    s = zc - 0.5 * jnp.sum(cb * cb, axis=-1)[None, :]
    idx_ref[...] = jnp.argmax(s, axis=-1).astype(jnp.int32)[:, None]


_SC_MESH = plsc.VectorSubcoreMesh(core_axis_name="c", subcore_axis_name="s")


@functools.partial(
    pl.kernel,
    mesh=_SC_MESH,
    out_type=jax.ShapeDtypeStruct((B, OUT_FEAT), jnp.float32),
    scratch_types=[
        pltpu.VMEM((N_CHUNKS, CHUNK), jnp.int32),
        pltpu.VMEM((CHUNK, OUT_FEAT), jnp.float32),
        pltpu.VMEM((CHUNK, OUT_FEAT), jnp.float32),
        pltpu.VMEM_SHARED((NUM_CODES, OUT_FEAT), jnp.float32),
        pltpu.SemaphoreType.DMA,
        pltpu.SemaphoreType.DMA,
    ],
)
def _sc_gather(table_hbm, idx_hbm, out_hbm, idx_v, rows_v0, rows_v1,
               table_sh, sem0, sem1):
    @pl.when(lax.axis_index("s") == 0)
    def _stage():
        pltpu.sync_copy(table_hbm, table_sh)

    plsc.subcore_barrier()
    wid = lax.axis_index("s") * NUM_CORES + lax.axis_index("c")
    base = wid * B_PER_WORKER
    pltpu.sync_copy(idx_hbm.at[pl.ds(wid * N_CHUNKS, N_CHUNKS)], idx_v)
    bufs = (rows_v0, rows_v1)
    sems = (sem0, sem1)
    pending = pltpu.async_copy(table_sh.at[idx_v.at[0]], bufs[0], sems[0])
    for c in range(N_CHUNKS):
        pending.wait()
        if c + 1 < N_CHUNKS:
            pending = pltpu.async_copy(table_sh.at[idx_v.at[c + 1]],
                                       bufs[(c + 1) % 2], sems[(c + 1) % 2])
        pltpu.sync_copy(bufs[c % 2], out_hbm.at[pl.ds(base + c * CHUNK, CHUNK)])


def kernel(x, W_down, b_down, W_in, b_in, codebook, W_out, b_out, W_up, b_up):
    table = pl.pallas_call(
        _fold_kernel,
        out_shape=jax.ShapeDtypeStruct((NUM_CODES, OUT_FEAT), jnp.float32),
    )(codebook, W_out, b_out, W_up, b_up)

    grid = (B // BLOCK_B,)
    idx2d = pl.pallas_call(
        _idx_kernel,
        grid=grid,
        in_specs=[
            pl.BlockSpec((BLOCK_B, IN_FEAT), lambda i: (i, 0)),
            pl.BlockSpec((IN_FEAT, HIDDEN), lambda i: (0, 0)),
            pl.BlockSpec((HIDDEN,), lambda i: (0,)),
            pl.BlockSpec((HIDDEN, CODEBOOK_DIM), lambda i: (0, 0)),
            pl.BlockSpec((CODEBOOK_DIM,), lambda i: (0,)),
            pl.BlockSpec((NUM_CODES, CODEBOOK_DIM), lambda i: (0, 0)),
        ],
        out_specs=pl.BlockSpec((BLOCK_B, 1), lambda i: (i, 0)),
        out_shape=jax.ShapeDtypeStruct((B, 1), jnp.int32),
        compiler_params=pltpu.CompilerParams(
            dimension_semantics=("parallel",),
        ),
    )(x, W_down, b_down, W_in, b_in, codebook)

    indices = idx2d.reshape(B)
    out = _sc_gather(table, idx2d.reshape(B // CHUNK, CHUNK))
    commit_loss = jnp.zeros((), jnp.float32)
    return out, indices, commit_loss


# speedup vs baseline: 11.7330x; 1.4647x over previous
import jax
import jax.numpy as jnp
from jax.experimental import pallas as pl
from jax.experimental.pallas import tpu as pltpu

B = 65536
IN_FEAT = 128
HIDDEN = 128
OUT_FEAT = 128
CODEBOOK_DIM = 32
NUM_CODES = 256

BLOCK_B = 8192


def _fold_kernel(cb_ref, wo_ref, bo_ref, wu_ref, bu_ref, table_ref):
    t = jnp.dot(cb_ref[...], wo_ref[...],
                preferred_element_type=jnp.float32) + bo_ref[...]
    t = jnp.dot(t, wu_ref[...],
                preferred_element_type=jnp.float32) + bu_ref[...]
    table_ref[...] = jnp.clip(t, -1.0, 1.0)


def _main_kernel(x_ref, wd_ref, bd_ref, wi_ref, bi_ref, cb_ref, table_ref,
                 out_ref, idx_ref):
    h = jnp.dot(x_ref[...], wd_ref[...],
                preferred_element_type=jnp.float32) + bd_ref[...]
    z = jnp.dot(h, wi_ref[...],
                preferred_element_type=jnp.float32) + bi_ref[...]
    cb = cb_ref[...]
    zc = jnp.dot(z, cb.T, preferred_element_type=jnp.float32)
    s = zc - 0.5 * jnp.sum(cb * cb, axis=-1)[None, :]
    idx = jnp.argmax(s, axis=-1).astype(jnp.int32)
    idx_ref[...] = idx[:, None]
    onehot = (jax.lax.broadcasted_iota(jnp.int32, (BLOCK_B, NUM_CODES), 1)
              == idx[:, None]).astype(jnp.float32)
    out_ref[...] = jnp.dot(onehot, table_ref[...],
                           preferred_element_type=jnp.float32)


def kernel(x, W_down, b_down, W_in, b_in, codebook, W_out, b_out, W_up, b_up):
    table = pl.pallas_call(
        _fold_kernel,
        out_shape=jax.ShapeDtypeStruct((NUM_CODES, OUT_FEAT), jnp.float32),
    )(codebook, W_out, b_out, W_up, b_up)

    grid = (B // BLOCK_B,)
    out, idx2d = pl.pallas_call(
        _main_kernel,
        grid=grid,
        in_specs=[
            pl.BlockSpec((BLOCK_B, IN_FEAT), lambda i: (i, 0)),
            pl.BlockSpec((IN_FEAT, HIDDEN), lambda i: (0, 0)),
            pl.BlockSpec((HIDDEN,), lambda i: (0,)),
            pl.BlockSpec((HIDDEN, CODEBOOK_DIM), lambda i: (0, 0)),
            pl.BlockSpec((CODEBOOK_DIM,), lambda i: (0,)),
            pl.BlockSpec((NUM_CODES, CODEBOOK_DIM), lambda i: (0, 0)),
            pl.BlockSpec((NUM_CODES, OUT_FEAT), lambda i: (0, 0)),
        ],
        out_specs=(
            pl.BlockSpec((BLOCK_B, OUT_FEAT), lambda i: (i, 0)),
            pl.BlockSpec((BLOCK_B, 1), lambda i: (i, 0)),
        ),
        out_shape=(
            jax.ShapeDtypeStruct((B, OUT_FEAT), jnp.float32),
            jax.ShapeDtypeStruct((B, 1), jnp.int32),
        ),
        compiler_params=pltpu.CompilerParams(
            dimension_semantics=("parallel",),
        ),
    )(x, W_down, b_down, W_in, b_in, codebook, table)

    indices = idx2d.reshape(B)
    commit_loss = jnp.zeros((), jnp.float32)
    return out, indices, commit_loss


# speedup vs baseline: 12.0170x; 1.0242x over previous
import jax
import jax.numpy as jnp
from jax.experimental import pallas as pl
from jax.experimental.pallas import tpu as pltpu

B = 65536
IN_FEAT = 128
HIDDEN = 128
OUT_FEAT = 128
CODEBOOK_DIM = 32
NUM_CODES = 256

BLOCK_B = 8192


def _main_kernel(x_ref, wd_ref, bd_ref, wi_ref, bi_ref, cb_ref, wo_ref,
                 bo_ref, wu_ref, bu_ref, out_ref, idx_ref):
    t = jnp.dot(cb_ref[...], wo_ref[...],
                preferred_element_type=jnp.float32) + bo_ref[...]
    t = jnp.dot(t, wu_ref[...],
                preferred_element_type=jnp.float32) + bu_ref[...]
    table = jnp.clip(t, -1.0, 1.0)
    h = jnp.dot(x_ref[...], wd_ref[...],
                preferred_element_type=jnp.float32) + bd_ref[...]
    z = jnp.dot(h, wi_ref[...],
                preferred_element_type=jnp.float32) + bi_ref[...]
    cb = cb_ref[...]
    zc = jnp.dot(z, cb.T, preferred_element_type=jnp.float32)
    s = zc - 0.5 * jnp.sum(cb * cb, axis=-1)[None, :]
    idx = jnp.argmax(s, axis=-1).astype(jnp.int32)
    idx_ref[...] = idx[:, None]
    onehot = (jax.lax.broadcasted_iota(jnp.int32, (BLOCK_B, NUM_CODES), 1)
              == idx[:, None]).astype(jnp.float32)
    out_ref[...] = jnp.dot(onehot, table,
                           preferred_element_type=jnp.float32)


def kernel(x, W_down, b_down, W_in, b_in, codebook, W_out, b_out, W_up, b_up):
    grid = (B // BLOCK_B,)
    out, idx2d = pl.pallas_call(
        _main_kernel,
        grid=grid,
        in_specs=[
            pl.BlockSpec((BLOCK_B, IN_FEAT), lambda i: (i, 0)),
            pl.BlockSpec((IN_FEAT, HIDDEN), lambda i: (0, 0)),
            pl.BlockSpec((HIDDEN,), lambda i: (0,)),
            pl.BlockSpec((HIDDEN, CODEBOOK_DIM), lambda i: (0, 0)),
            pl.BlockSpec((CODEBOOK_DIM,), lambda i: (0,)),
            pl.BlockSpec((NUM_CODES, CODEBOOK_DIM), lambda i: (0, 0)),
            pl.BlockSpec((CODEBOOK_DIM, HIDDEN), lambda i: (0, 0)),
            pl.BlockSpec((HIDDEN,), lambda i: (0,)),
            pl.BlockSpec((HIDDEN, OUT_FEAT), lambda i: (0, 0)),
            pl.BlockSpec((OUT_FEAT,), lambda i: (0,)),
        ],
        out_specs=(
            pl.BlockSpec((BLOCK_B, OUT_FEAT), lambda i: (i, 0)),
            pl.BlockSpec((BLOCK_B, 1), lambda i: (i, 0)),
        ),
        out_shape=(
            jax.ShapeDtypeStruct((B, OUT_FEAT), jnp.float32),
            jax.ShapeDtypeStruct((B, 1), jnp.int32),
        ),
        compiler_params=pltpu.CompilerParams(
            dimension_semantics=("parallel",),
        ),
    )(x, W_down, b_down, W_in, b_in, codebook, W_out, b_out, W_up, b_up)

    indices = idx2d.reshape(B)
    commit_loss = jnp.zeros((), jnp.float32)
    return out, indices, commit_loss
